# Initial kernel scaffold; baseline (speedup 1.0000x reference)
import jax
import jax.numpy as jnp
from jax.experimental import pallas as pl


def kernel(x, edge_index, W_l, b_l, W_r, b_r, att, bias, gamma, beta):
    raise NotImplementedError("write your pallas kernel here")


# speedup vs baseline: 36.7535x; 36.7535x over previous
import functools

import jax
import jax.numpy as jnp
from jax import lax
from jax.experimental import pallas as pl
from jax.experimental.pallas import tpu as pltpu
from jax.experimental.pallas import tpu_sc as plsc

NEG_SLOPE = 0.2
NW = 32
CH = 128


def _mm_body(x_ref, wl_ref, bl_ref, wr_ref, br_ref, xl_ref, xr_ref):
    x = x_ref[...]
    xl_ref[...] = jnp.dot(x, wl_ref[...], preferred_element_type=jnp.float32) + bl_ref[...]
    xr_ref[...] = jnp.dot(x, wr_ref[...], preferred_element_type=jnp.float32) + br_ref[...]


def _transforms(x, W_l, b_l, W_r, b_r, BN=1000):
    N, D = x.shape
    grid = N // BN
    return pl.pallas_call(
        _mm_body,
        grid=(grid,),
        in_specs=[
            pl.BlockSpec((BN, D), lambda i: (i, 0)),
            pl.BlockSpec((D, D), lambda i: (0, 0)),
            pl.BlockSpec((1, D), lambda i: (0, 0)),
            pl.BlockSpec((D, D), lambda i: (0, 0)),
            pl.BlockSpec((1, D), lambda i: (0, 0)),
        ],
        out_specs=[
            pl.BlockSpec((BN, D), lambda i: (i, 0)),
            pl.BlockSpec((BN, D), lambda i: (i, 0)),
        ],
        out_shape=[
            jax.ShapeDtypeStruct((N, D), jnp.float32),
            jax.ShapeDtypeStruct((N, D), jnp.float32),
        ],
    )(x, W_l, b_l.reshape(1, D), W_r, b_r.reshape(1, D))


def _make_gather(Mp, N, D, nchunk):
    mesh = plsc.VectorSubcoreMesh(core_axis_name="c", subcore_axis_name="s")

    @functools.partial(
        pl.kernel,
        mesh=mesh,
        out_type=[
            jax.ShapeDtypeStruct((Mp, D), jnp.float32),
            jax.ShapeDtypeStruct((Mp, D), jnp.float32),
        ],
        scratch_types=[
            pltpu.VMEM((nchunk, CH), jnp.int32),
            pltpu.VMEM((nchunk, CH), jnp.int32),
            pltpu.VMEM((CH, D), jnp.float32),
            pltpu.VMEM((CH, D), jnp.float32),
            pltpu.SemaphoreType.DMA,
            pltpu.SemaphoreType.DMA,
        ],
    )
    def gather_k(xl_hbm, xr_hbm, src2_hbm, dstg2_hbm, xlg_out, xrg_out,
                 idx_s, idx_d, bufl, bufr, semL, semR):
        wid = lax.axis_index("s") * 2 + lax.axis_index("c")
        rowbase = wid * nchunk
        pltpu.sync_copy(src2_hbm.at[wid], idx_s)
        pltpu.sync_copy(dstg2_hbm.at[wid], idx_d)

        def step(j, _):
            cl = pltpu.async_copy(xl_hbm.at[idx_s.at[j]], bufl, semL)
            cr = pltpu.async_copy(xr_hbm.at[idx_d.at[j]], bufr, semR)
            cl.wait()
            cr.wait()
            mbase = (rowbase + j) * CH
            pltpu.sync_copy(bufl, xlg_out.at[pl.ds(mbase, CH)])
            pltpu.sync_copy(bufr, xrg_out.at[pl.ds(mbase, CH)])
            return 0

        lax.fori_loop(0, nchunk, step, 0)

    return gather_k


def _logits_body(xlg_ref, xrg_ref, attf_ref, g_ref, lg_ref, gmax_ref):
    i = pl.program_id(0)
    s = xlg_ref[...] + xrg_ref[...]
    s = jnp.where(s >= 0, s, NEG_SLOPE * s) * attf_ref[...]
    lg = jnp.dot(s, g_ref[...], preferred_element_type=jnp.float32)
    lg_ref[...] = lg

    @pl.when(i == 0)
    def _():
        gmax_ref[...] = jnp.full_like(gmax_ref, -jnp.inf)

    gmax_ref[...] = jnp.maximum(gmax_ref[...], jnp.max(lg))


def _logits(xlg, xrg, attf, G, BM=2048):
    Mp, D = xlg.shape
    H = G.shape[1]
    return pl.pallas_call(
        _logits_body,
        grid=(Mp // BM,),
        in_specs=[
            pl.BlockSpec((BM, D), lambda i: (i, 0)),
            pl.BlockSpec((BM, D), lambda i: (i, 0)),
            pl.BlockSpec((1, D), lambda i: (0, 0)),
            pl.BlockSpec((D, H), lambda i: (0, 0)),
        ],
        out_specs=[
            pl.BlockSpec((BM, H), lambda i: (i, 0)),
            pl.BlockSpec((1, D), lambda i: (0, 0)),
        ],
        out_shape=[
            jax.ShapeDtypeStruct((Mp, H), jnp.float32),
            jax.ShapeDtypeStruct((1, D), jnp.float32),
        ],
    )(xlg, xrg, attf, G)


def _weights_body(xlg_ref, lg_ref, gmax_ref, ehd_ref, w_ref, pe_ref):
    p = jnp.exp(lg_ref[...] - jnp.max(gmax_ref[...]))
    pe = jnp.dot(p, ehd_ref[...], preferred_element_type=jnp.float32)
    pe_ref[...] = pe
    w_ref[...] = xlg_ref[...] * pe


def _weights(xlg, lg, gmax, Ehd, BM=2048):
    Mp, D = xlg.shape
    H = lg.shape[1]
    return pl.pallas_call(
        _weights_body,
        grid=(Mp // BM,),
        in_specs=[
            pl.BlockSpec((BM, D), lambda i: (i, 0)),
            pl.BlockSpec((BM, H), lambda i: (i, 0)),
            pl.BlockSpec((1, D), lambda i: (0, 0)),
            pl.BlockSpec((H, D), lambda i: (0, 0)),
        ],
        out_specs=[
            pl.BlockSpec((BM, D), lambda i: (i, 0)),
            pl.BlockSpec((BM, D), lambda i: (i, 0)),
        ],
        out_shape=[
            jax.ShapeDtypeStruct((Mp, D), jnp.float32),
            jax.ShapeDtypeStruct((Mp, D), jnp.float32),
        ],
    )(xlg, lg, gmax, Ehd)


def _make_scatter(Mp, Np, D, nchunk):
    mesh = plsc.VectorSubcoreMesh(core_axis_name="c", subcore_axis_name="s")
    stripe = Np // 16
    nz = stripe // CH

    @functools.partial(
        pl.kernel,
        mesh=mesh,
        out_type=[
            jax.ShapeDtypeStruct((Np, D), jnp.float32),
            jax.ShapeDtypeStruct((Np, D), jnp.float32),
        ],
        scratch_types=[
            pltpu.VMEM((nchunk, CH), jnp.int32),
            pltpu.VMEM((CH, D), jnp.float32),
            pltpu.VMEM_SHARED((Np, D), jnp.float32),
            pltpu.SemaphoreType.DMA,
        ],
    )
    def scatter_k(w_hbm, pe_hbm, dsts2_hbm, acc_out, den_out,
                  idx_d, wbuf, tab_sh, sem):
        cid = lax.axis_index("c")
        sid = lax.axis_index("s")
        rowbase = sid * nchunk
        pltpu.sync_copy(dsts2_hbm.at[sid], idx_d)

        def zrow(i, _):
            def zcol(c, _):
                wbuf[i, pl.ds(c * 16, 16)] = jnp.zeros((16,), jnp.float32)
                return 0
            lax.fori_loop(0, D // 16, zcol, 0)
            return 0
        lax.fori_loop(0, CH, zrow, 0)

        def zstripe(k, _):
            pltpu.sync_copy(wbuf, tab_sh.at[pl.ds(sid * stripe + k * CH, CH)])
            return 0
        lax.fori_loop(0, nz, zstripe, 0)
        plsc.subcore_barrier()

        def step_from(src_hbm):
            def step(j, _):
                pltpu.sync_copy(src_hbm.at[pl.ds((rowbase + j) * CH, CH)], wbuf)
                pltpu.sync_copy(wbuf, tab_sh.at[idx_d.at[j]], add=True)
                return 0
            return step

        @pl.when(cid == 0)
        def _():
            lax.fori_loop(0, nchunk, step_from(w_hbm), 0)

        @pl.when(cid == 1)
        def _():
            lax.fori_loop(0, nchunk, step_from(pe_hbm), 0)

        plsc.subcore_barrier()

        def dump_to(dst_hbm):
            def dump(k, _):
                off = sid * stripe + k * CH
                pltpu.sync_copy(tab_sh.at[pl.ds(off, CH)],
                                dst_hbm.at[pl.ds(off, CH)])
                return 0
            return dump

        @pl.when(cid == 0)
        def _():
            lax.fori_loop(0, nz, dump_to(acc_out), 0)

        @pl.when(cid == 1)
        def _():
            lax.fori_loop(0, nz, dump_to(den_out), 0)

    return scatter_k


def _final_body(x_ref, acc_ref, den_ref, bias_ref, gamma_ref, beta_ref, out_ref):
    y = x_ref[...] + acc_ref[...] / (den_ref[...] + 1e-16) + bias_ref[...]
    mu = jnp.mean(y, axis=1, keepdims=True)
    yc = y - mu
    var = jnp.mean(yc * yc, axis=1, keepdims=True)
    out_ref[...] = gamma_ref[...] * yc * jax.lax.rsqrt(var + 1e-5) + beta_ref[...]


def _finalize(x, acc, den, bias, gamma, beta, BN=1000):
    N, D = x.shape
    return pl.pallas_call(
        _final_body,
        grid=(N // BN,),
        in_specs=[
            pl.BlockSpec((BN, D), lambda i: (i, 0)),
            pl.BlockSpec((BN, D), lambda i: (i, 0)),
            pl.BlockSpec((BN, D), lambda i: (i, 0)),
            pl.BlockSpec((1, D), lambda i: (0, 0)),
            pl.BlockSpec((1, D), lambda i: (0, 0)),
            pl.BlockSpec((1, D), lambda i: (0, 0)),
        ],
        out_specs=pl.BlockSpec((BN, D), lambda i: (i, 0)),
        out_shape=jax.ShapeDtypeStruct((N, D), jnp.float32),
    )(x, acc, den, bias.reshape(1, D), gamma.reshape(1, D), beta.reshape(1, D))


def kernel(x, edge_index, W_l, b_l, W_r, b_r, att, bias, gamma, beta):
    N, D = x.shape
    H, C = att.shape
    E = edge_index.shape[1]
    M = E + N
    Mp = ((M + NW * CH - 1) // (NW * CH)) * (NW * CH)
    nchunk = Mp // (NW * CH)
    nchunk_s = Mp // (16 * CH)
    Np = ((N + 1 + 16 * CH - 1) // (16 * CH)) * (16 * CH)

    sl = jnp.arange(N, dtype=edge_index.dtype)
    src = jnp.concatenate([edge_index[0], sl])
    dst = jnp.concatenate([edge_index[1], sl])
    pad = Mp - M
    src_p = jnp.pad(src, (0, pad)).reshape(NW, nchunk, CH)
    dstg_p = jnp.pad(dst, (0, pad)).reshape(NW, nchunk, CH)
    dsts_p = jnp.pad(dst, (0, pad), constant_values=N).reshape(16, nchunk_s, CH)

    eyeH = jnp.eye(H, dtype=jnp.float32)
    Ehd = jnp.repeat(eyeH, C, axis=1)
    G = Ehd.T
    attf = att.reshape(1, H * C)

    xl, xr = _transforms(x, W_l, b_l, W_r, b_r)
    xlg, xrg = _make_gather(Mp, N, D, nchunk)(xl, xr, src_p, dstg_p)
    lg, gmax = _logits(xlg, xrg, attf, G)
    w, pe = _weights(xlg, lg, gmax, Ehd)
    acc, den = _make_scatter(Mp, Np, D, nchunk_s)(w, pe, dsts_p)
    return _finalize(x, acc[:N], den[:N], bias, gamma, beta)


# speedup vs baseline: 43.9815x; 1.1967x over previous
import functools

import jax
import jax.numpy as jnp
from jax import lax
from jax.experimental import pallas as pl
from jax.experimental.pallas import tpu as pltpu
from jax.experimental.pallas import tpu_sc as plsc

NEG_SLOPE = 0.2
NW = 32
CH = 128


def _mm_body(x_ref, wl_ref, bl_ref, wr_ref, br_ref, xl_ref, xr_ref):
    x = x_ref[...]
    xl_ref[...] = jnp.dot(x, wl_ref[...], preferred_element_type=jnp.float32) + bl_ref[...]
    xr_ref[...] = jnp.dot(x, wr_ref[...], preferred_element_type=jnp.float32) + br_ref[...]


def _transforms(x, W_l, b_l, W_r, b_r, BN=1000):
    N, D = x.shape
    grid = N // BN
    return pl.pallas_call(
        _mm_body,
        grid=(grid,),
        in_specs=[
            pl.BlockSpec((BN, D), lambda i: (i, 0)),
            pl.BlockSpec((D, D), lambda i: (0, 0)),
            pl.BlockSpec((1, D), lambda i: (0, 0)),
            pl.BlockSpec((D, D), lambda i: (0, 0)),
            pl.BlockSpec((1, D), lambda i: (0, 0)),
        ],
        out_specs=[
            pl.BlockSpec((BN, D), lambda i: (i, 0)),
            pl.BlockSpec((BN, D), lambda i: (i, 0)),
        ],
        out_shape=[
            jax.ShapeDtypeStruct((N, D), jnp.float32),
            jax.ShapeDtypeStruct((N, D), jnp.float32),
        ],
    )(x, W_l, b_l.reshape(1, D), W_r, b_r.reshape(1, D))


def _make_gather(Mp, N, D, nchunk):
    mesh = plsc.VectorSubcoreMesh(core_axis_name="c", subcore_axis_name="s")

    @functools.partial(
        pl.kernel,
        mesh=mesh,
        out_type=[
            jax.ShapeDtypeStruct((Mp, D), jnp.float32),
            jax.ShapeDtypeStruct((Mp, D), jnp.float32),
        ],
        scratch_types=[
            pltpu.VMEM((nchunk, CH), jnp.int32),
            pltpu.VMEM((nchunk, CH), jnp.int32),
            pltpu.VMEM((CH, D), jnp.float32),
            pltpu.VMEM((CH, D), jnp.float32),
            pltpu.VMEM((CH, D), jnp.float32),
            pltpu.VMEM((CH, D), jnp.float32),
            pltpu.SemaphoreType.DMA,
            pltpu.SemaphoreType.DMA,
            pltpu.SemaphoreType.DMA,
            pltpu.SemaphoreType.DMA,
        ],
    )
    def gather_k(xl_hbm, xr_hbm, src2_hbm, dstg2_hbm, xlg_out, xrg_out,
                 idx_s, idx_d, bufl0, bufr0, bufl1, bufr1,
                 semL0, semR0, semL1, semR1):
        wid = lax.axis_index("s") * 2 + lax.axis_index("c")
        rowbase = wid * nchunk
        pltpu.sync_copy(src2_hbm.at[wid], idx_s)
        pltpu.sync_copy(dstg2_hbm.at[wid], idx_d)

        bufs = ((bufl0, bufr0, semL0, semR0), (bufl1, bufr1, semL1, semR1))

        def issue(j, b):
            bl, br, sl, sr = bufs[b]
            pltpu.async_copy(xl_hbm.at[idx_s.at[j]], bl, sl)
            pltpu.async_copy(xr_hbm.at[idx_d.at[j]], br, sr)

        def wait_g(b):
            bl, br, sl, sr = bufs[b]
            pltpu.make_async_copy(xl_hbm.at[pl.ds(0, CH)], bl, sl).wait()
            pltpu.make_async_copy(xr_hbm.at[pl.ds(0, CH)], br, sr).wait()

        def drain(j, b):
            bl, br, _, _ = bufs[b]
            wait_g(b)
            mbase = (rowbase + j) * CH
            pltpu.sync_copy(bl, xlg_out.at[pl.ds(mbase, CH)])
            pltpu.sync_copy(br, xrg_out.at[pl.ds(mbase, CH)])

        issue(0, 0)
        issue(1, 1)

        def body2(j2, _):
            j = j2 * 2
            drain(j, 0)

            @pl.when(j + 2 < nchunk)
            def _():
                issue(j + 2, 0)

            drain(j + 1, 1)

            @pl.when(j + 3 < nchunk)
            def _():
                issue(j + 3, 1)
            return 0

        lax.fori_loop(0, nchunk // 2, body2, 0)
        if nchunk % 2:
            drain(nchunk - 1, (nchunk - 1) % 2)

    return gather_k


def _logits_body(xlg_ref, xrg_ref, attf_ref, g_ref, lg_ref, gmax_ref):
    i = pl.program_id(0)
    s = xlg_ref[...] + xrg_ref[...]
    s = jnp.where(s >= 0, s, NEG_SLOPE * s) * attf_ref[...]
    lg = jnp.dot(s, g_ref[...], preferred_element_type=jnp.float32)
    lg_ref[...] = lg

    @pl.when(i == 0)
    def _():
        gmax_ref[...] = jnp.full_like(gmax_ref, -jnp.inf)

    gmax_ref[...] = jnp.maximum(gmax_ref[...], jnp.max(lg))


def _logits(xlg, xrg, attf, G, BM=2048):
    Mp, D = xlg.shape
    H = G.shape[1]
    return pl.pallas_call(
        _logits_body,
        grid=(Mp // BM,),
        in_specs=[
            pl.BlockSpec((BM, D), lambda i: (i, 0)),
            pl.BlockSpec((BM, D), lambda i: (i, 0)),
            pl.BlockSpec((1, D), lambda i: (0, 0)),
            pl.BlockSpec((D, H), lambda i: (0, 0)),
        ],
        out_specs=[
            pl.BlockSpec((BM, H), lambda i: (i, 0)),
            pl.BlockSpec((1, D), lambda i: (0, 0)),
        ],
        out_shape=[
            jax.ShapeDtypeStruct((Mp, H), jnp.float32),
            jax.ShapeDtypeStruct((1, D), jnp.float32),
        ],
    )(xlg, xrg, attf, G)


def _weights_body(xlg_ref, lg_ref, gmax_ref, ehd_ref, w_ref, pe_ref):
    p = jnp.exp(lg_ref[...] - jnp.max(gmax_ref[...]))
    pe = jnp.dot(p, ehd_ref[...], preferred_element_type=jnp.float32)
    pe_ref[...] = pe
    w_ref[...] = xlg_ref[...] * pe


def _weights(xlg, lg, gmax, Ehd, BM=2048):
    Mp, D = xlg.shape
    H = lg.shape[1]
    return pl.pallas_call(
        _weights_body,
        grid=(Mp // BM,),
        in_specs=[
            pl.BlockSpec((BM, D), lambda i: (i, 0)),
            pl.BlockSpec((BM, H), lambda i: (i, 0)),
            pl.BlockSpec((1, D), lambda i: (0, 0)),
            pl.BlockSpec((H, D), lambda i: (0, 0)),
        ],
        out_specs=[
            pl.BlockSpec((BM, D), lambda i: (i, 0)),
            pl.BlockSpec((BM, D), lambda i: (i, 0)),
        ],
        out_shape=[
            jax.ShapeDtypeStruct((Mp, D), jnp.float32),
            jax.ShapeDtypeStruct((Mp, D), jnp.float32),
        ],
    )(xlg, lg, gmax, Ehd)


def _make_scatter(Mp, Np, D, nchunk):
    mesh = plsc.VectorSubcoreMesh(core_axis_name="c", subcore_axis_name="s")
    stripe = Np // 16
    nz = stripe // CH

    @functools.partial(
        pl.kernel,
        mesh=mesh,
        out_type=[
            jax.ShapeDtypeStruct((Np, D), jnp.float32),
            jax.ShapeDtypeStruct((Np, D), jnp.float32),
        ],
        scratch_types=[
            pltpu.VMEM((1, CH), jnp.int32),
            pltpu.VMEM((1, CH), jnp.int32),
            pltpu.VMEM((CH, D), jnp.float32),
            pltpu.VMEM((CH, D), jnp.float32),
            pltpu.VMEM_SHARED((Np, D), jnp.float32),
            pltpu.SemaphoreType.DMA,
            pltpu.SemaphoreType.DMA,
            pltpu.SemaphoreType.DMA,
            pltpu.SemaphoreType.DMA,
        ],
    )
    def scatter_k(w_hbm, pe_hbm, dsts2_hbm, acc_out, den_out,
                  idxb0, idxb1, wbuf, wbuf1, tab_sh, sem, sem1, semI0, semI1):
        cid = lax.axis_index("c")
        sid = lax.axis_index("s")
        rowbase = sid * nchunk

        def zrow(i, _):
            def zcol(c, _):
                wbuf[i, pl.ds(c * 16, 16)] = jnp.zeros((16,), jnp.float32)
                return 0
            lax.fori_loop(0, D // 16, zcol, 0)
            return 0
        lax.fori_loop(0, CH, zrow, 0)

        def zstripe(k, _):
            pltpu.sync_copy(wbuf, tab_sh.at[pl.ds(sid * stripe + k * CH, CH)])
            return 0
        lax.fori_loop(0, nz, zstripe, 0)
        plsc.subcore_barrier()

        bufs = ((wbuf, sem, idxb0, semI0), (wbuf1, sem1, idxb1, semI1))

        def run_from(src_hbm):
            def issue(j, b):
                bb, ss, ib, si = bufs[b]
                pltpu.async_copy(dsts2_hbm.at[sid, pl.ds(j, 1)], ib, si)
                pltpu.async_copy(src_hbm.at[pl.ds((rowbase + j) * CH, CH)], bb, ss)

            def drain(j, b):
                bb, ss, ib, si = bufs[b]
                pltpu.make_async_copy(dsts2_hbm.at[sid, pl.ds(0, 1)], ib, si).wait()
                pltpu.make_async_copy(src_hbm.at[pl.ds(0, CH)], bb, ss).wait()
                pltpu.sync_copy(bb, tab_sh.at[ib.at[0]], add=True)

            issue(0, 0)
            issue(1, 1)

            def body2(j2, _):
                j = j2 * 2
                drain(j, 0)

                @pl.when(j + 2 < nchunk)
                def _():
                    issue(j + 2, 0)

                drain(j + 1, 1)

                @pl.when(j + 3 < nchunk)
                def _():
                    issue(j + 3, 1)
                return 0

            lax.fori_loop(0, nchunk // 2, body2, 0)
            if nchunk % 2:
                drain(nchunk - 1, (nchunk - 1) % 2)

        @pl.when(cid == 0)
        def _():
            run_from(w_hbm)

        @pl.when(cid == 1)
        def _():
            run_from(pe_hbm)

        plsc.subcore_barrier()

        def dump_to(dst_hbm):
            def dump(k, _):
                off = sid * stripe + k * CH
                pltpu.sync_copy(tab_sh.at[pl.ds(off, CH)],
                                dst_hbm.at[pl.ds(off, CH)])
                return 0
            return dump

        @pl.when(cid == 0)
        def _():
            lax.fori_loop(0, nz, dump_to(acc_out), 0)

        @pl.when(cid == 1)
        def _():
            lax.fori_loop(0, nz, dump_to(den_out), 0)

    return scatter_k


def _final_body(x_ref, acc_ref, den_ref, bias_ref, gamma_ref, beta_ref, out_ref):
    y = x_ref[...] + acc_ref[...] / (den_ref[...] + 1e-16) + bias_ref[...]
    mu = jnp.mean(y, axis=1, keepdims=True)
    yc = y - mu
    var = jnp.mean(yc * yc, axis=1, keepdims=True)
    out_ref[...] = gamma_ref[...] * yc * jax.lax.rsqrt(var + 1e-5) + beta_ref[...]


def _finalize(x, acc, den, bias, gamma, beta, BN=1000):
    N, D = x.shape
    return pl.pallas_call(
        _final_body,
        grid=(N // BN,),
        in_specs=[
            pl.BlockSpec((BN, D), lambda i: (i, 0)),
            pl.BlockSpec((BN, D), lambda i: (i, 0)),
            pl.BlockSpec((BN, D), lambda i: (i, 0)),
            pl.BlockSpec((1, D), lambda i: (0, 0)),
            pl.BlockSpec((1, D), lambda i: (0, 0)),
            pl.BlockSpec((1, D), lambda i: (0, 0)),
        ],
        out_specs=pl.BlockSpec((BN, D), lambda i: (i, 0)),
        out_shape=jax.ShapeDtypeStruct((N, D), jnp.float32),
    )(x, acc, den, bias.reshape(1, D), gamma.reshape(1, D), beta.reshape(1, D))


def kernel(x, edge_index, W_l, b_l, W_r, b_r, att, bias, gamma, beta):
    N, D = x.shape
    H, C = att.shape
    E = edge_index.shape[1]
    M = E + N
    Mp = ((M + NW * CH - 1) // (NW * CH)) * (NW * CH)
    nchunk = Mp // (NW * CH)
    nchunk_s = Mp // (16 * CH)
    Np = ((N + 1 + 16 * CH - 1) // (16 * CH)) * (16 * CH)

    sl = jnp.arange(N, dtype=edge_index.dtype)
    src = jnp.concatenate([edge_index[0], sl])
    dst = jnp.concatenate([edge_index[1], sl])
    pad = Mp - M
    src_p = jnp.pad(src, (0, pad)).reshape(NW, nchunk, CH)
    dstg_p = jnp.pad(dst, (0, pad)).reshape(NW, nchunk, CH)
    dsts_p = jnp.pad(dst, (0, pad), constant_values=N).reshape(16, nchunk_s, CH)

    eyeH = jnp.eye(H, dtype=jnp.float32)
    Ehd = jnp.repeat(eyeH, C, axis=1)
    G = Ehd.T
    attf = att.reshape(1, H * C)

    xl, xr = _transforms(x, W_l, b_l, W_r, b_r)
    xlg, xrg = _make_gather(Mp, N, D, nchunk)(xl, xr, src_p, dstg_p)
    lg, gmax = _logits(xlg, xrg, attf, G)
    w, pe = _weights(xlg, lg, gmax, Ehd)
    acc, den = _make_scatter(Mp, Np, D, nchunk_s)(w, pe, dsts_p)
    return _finalize(x, acc[:N], den[:N], bias, gamma, beta)
